# baseline (speedup 1.0000x reference)
import jax
import jax.numpy as jnp
import numpy as np
from jax.experimental import pallas as pl

DIM = 32768
BATCH = 128
N_STEPS = 2
TEMP = 2.0
RANK = 64

BB = 16
CW = 2048
NC = DIM // CW
_TINY = np.float32(np.finfo(np.float32).tiny)
_NEG_INF = np.float32(-np.inf)


def _np_threefry(k1, k2, x0, x1):
    u = np.uint32
    rots = ((13, 15, 26, 6), (17, 29, 16, 24))
    ks = (u(k1), u(k2), u(k1) ^ u(k2) ^ u(0x1BD11BDA))
    x0 = (u(x0) + ks[0]).astype(u)
    x1 = (u(x1) + ks[1]).astype(u)
    for i in range(5):
        for r in rots[i % 2]:
            x0 = (x0 + x1).astype(u)
            x1 = ((x1 << u(r)) | (x1 >> u(32 - r))).astype(u)
            x1 = x1 ^ x0
        x0 = (x0 + ks[(i + 1) % 3]).astype(u)
        x1 = (x1 + ks[(i + 2) % 3] + u(i + 1)).astype(u)
    return x0, x1


def _fold_in_key(k1, k2, data):
    return _np_threefry(k1, k2, 0, data)


with np.errstate(over="ignore"):
    _KEYS = tuple(_fold_in_key(0, 42, 3 * i) for i in range(N_STEPS))


def _threefry_gumbel(k1, k2, cnt):
    u32 = jnp.uint32
    ks = (u32(k1), u32(k2), u32(k1) ^ u32(k2) ^ u32(0x1BD11BDA))
    rots = ((13, 15, 26, 6), (17, 29, 16, 24))
    x1 = cnt + ks[1]
    x0 = x1 + ks[0]
    first = True
    for i in range(5):
        for r in rots[i % 2]:
            if first:
                first = False
            else:
                x0 = x0 + x1
            x1 = (x1 << u32(r)) | (x1 >> u32(32 - r))
            x1 = x1 ^ x0
        x0 = x0 + ks[(i + 1) % 3]
        x1 = x1 + ks[(i + 2) % 3] + u32(i + 1)
    bits = x0 ^ x1
    fb = (bits >> u32(9)) | u32(0x3F800000)
    f = jax.lax.bitcast_convert_type(fb, jnp.float32) - 1.0
    v = jnp.maximum(_TINY, f * (np.float32(1.0) - _TINY) + _TINY)
    return -jnp.log(-jnp.log(v))


def _body(x_ref, b_ref, w_ref, u0_ref, u1_ref, o_ref):
    f32 = jnp.float32
    u32 = jnp.uint32
    us = (u0_ref, u1_ref)
    W = w_ref[...]
    col_iota = jax.lax.broadcasted_iota(jnp.int32, (BB, CW), 1)
    ucnt = (jax.lax.broadcasted_iota(u32, (BB, CW), 0) * u32(DIM)
            + jax.lax.broadcasted_iota(u32, (BB, CW), 1))
    base_count = (pl.program_id(0) * (BB * DIM)).astype(u32)

    xw = jnp.dot(x_ref[...], W, preferred_element_type=f32)

    for i in range(N_STEPS):
        cur = x_ref if i == 0 else o_ref
        k1, k2 = _KEYS[i]
        u = us[i][:, 0]

        def pass_b(c, carry):
            mx, S, bv, bidx = carry
            c0 = c * CW
            xc = cur[:, pl.ds(c0, CW)]
            sgn = 1.0 - 2.0 * xc
            wc = w_ref[pl.ds(c0, CW), :]
            gx = jax.lax.dot_general(xw, wc, (((1,), (1,)), ((), ())),
                                     preferred_element_type=f32)
            fd = sgn * (gx + b_ref[:, pl.ds(c0, CW)]) * 0.5
            cmx = jnp.max(fd, axis=-1)
            nmx = jnp.maximum(mx, cmx)
            S = S * jnp.exp(mx - nmx) + jnp.sum(
                jnp.exp(fd - nmx[:, None]), axis=-1)
            g = _threefry_gumbel(k1, k2, ucnt + (base_count + u32(c0)))
            m = fd + g
            cbv = jnp.max(m, axis=-1)
            cbi = jnp.argmax(m, axis=-1) + c0
            upd = cbv > bv
            bv = jnp.where(upd, cbv, bv)
            bidx = jnp.where(upd, cbi, bidx)
            return nmx, S, bv, bidx

        init = (jnp.full((BB,), _NEG_INF, f32), jnp.zeros((BB,), f32),
                jnp.full((BB,), _NEG_INF, f32), jnp.zeros((BB,), jnp.int32))
        mx, S, _, idx = jax.lax.fori_loop(0, NC, pass_b, init)
        lse_f = mx + jnp.log(S)

        def pass_g(c, carry):
            bi, s, Wi = carry
            c0 = c * CW
            mask = (col_iota + c0) == idx[:, None]
            xc = cur[:, pl.ds(c0, CW)]
            bi = bi + jnp.sum(jnp.where(mask, b_ref[:, pl.ds(c0, CW)], 0.0),
                              axis=-1)
            s = s + jnp.sum(jnp.where(mask, 1.0 - 2.0 * xc, 0.0), axis=-1)
            Wi = Wi + jnp.dot(mask.astype(f32), w_ref[pl.ds(c0, CW), :],
                              preferred_element_type=f32)
            return bi, s, Wi

        bi, s, Wi = jax.lax.fori_loop(
            0, NC, pass_g,
            (jnp.zeros((BB,), f32), jnp.zeros((BB,), f32),
             jnp.zeros((BB, RANK), f32)))

        xw_new = xw + s[:, None] * Wi
        fd_at = s * (jnp.sum(xw * Wi, axis=-1) + bi) * 0.5
        rd_at = -s * (jnp.sum(xw_new * Wi, axis=-1) + bi) * 0.5
        m_term = s * bi + 0.5 * (jnp.sum(xw_new * xw_new, axis=-1)
                                 - jnp.sum(xw * xw, axis=-1))

        def pass_c(c, carry):
            mxr, Sr = carry
            c0 = c * CW
            xc = cur[:, pl.ds(c0, CW)]
            t = 1.0 - 2.0 * xc
            mask = (col_iota + c0) == idx[:, None]
            sgn_new = jnp.where(mask, -t, t)
            wc = w_ref[pl.ds(c0, CW), :]
            gxn = jax.lax.dot_general(xw_new, wc, (((1,), (1,)), ((), ())),
                                      preferred_element_type=f32)
            rd = sgn_new * (gxn + b_ref[:, pl.ds(c0, CW)]) * 0.5
            cmx = jnp.max(rd, axis=-1)
            nmx = jnp.maximum(mxr, cmx)
            Sr = Sr * jnp.exp(mxr - nmx) + jnp.sum(
                jnp.exp(rd - nmx[:, None]), axis=-1)
            return nmx, Sr

        mxr, Sr = jax.lax.fori_loop(
            0, NC, pass_c,
            (jnp.full((BB,), _NEG_INF, f32), jnp.zeros((BB,), f32)))
        lse_r = mxr + jnp.log(Sr)

        la = m_term + (rd_at - lse_r) - (fd_at - lse_f)
        a = jnp.exp(la) > u
        xw = jnp.where(a[:, None], xw_new, xw)

        def pass_u(c, _):
            c0 = c * CW
            mask = (col_iota + c0) == idx[:, None]
            xc = cur[:, pl.ds(c0, CW)]
            o_ref[:, pl.ds(c0, CW)] = jnp.where(
                mask & a[:, None], 1.0 - xc, xc)
            return 0

        jax.lax.fori_loop(0, NC, pass_u, 0)


@jax.jit
def kernel(x, b, W):
    base = jax.random.key(42)
    us = []
    for i in range(N_STEPS):
        ka = jax.random.fold_in(base, 3 * i + 1)
        u = jax.random.uniform(ka, (BATCH,), jnp.float32)
        us.append(jnp.broadcast_to(u[:, None], (BATCH, 128)))
    b2 = b.reshape(1, DIM)

    grid = (BATCH // BB,)
    row = lambda i: (i, 0)
    fixed = lambda i: (0, 0)
    return pl.pallas_call(
        _body,
        grid=grid,
        in_specs=[
            pl.BlockSpec((BB, DIM), row),
            pl.BlockSpec((1, DIM), fixed),
            pl.BlockSpec((DIM, RANK), fixed),
            pl.BlockSpec((BB, 128), row),
            pl.BlockSpec((BB, 128), row),
        ],
        out_specs=pl.BlockSpec((BB, DIM), row),
        out_shape=jax.ShapeDtypeStruct((BATCH, DIM), x.dtype),
    )(x, b2, W, us[0], us[1])


# speedup vs baseline: 1.4511x; 1.4511x over previous
import jax
import jax.numpy as jnp
import numpy as np
from jax.experimental import pallas as pl

DIM = 32768
BATCH = 128
N_STEPS = 2
TEMP = 2.0
RANK = 64

BB = 16
CW = 4096
NC = DIM // CW
_TINY = np.float32(np.finfo(np.float32).tiny)
_NEG_INF = np.float32(-np.inf)


def _np_threefry(k1, k2, x0, x1):
    u = np.uint32
    rots = ((13, 15, 26, 6), (17, 29, 16, 24))
    ks = (u(k1), u(k2), u(k1) ^ u(k2) ^ u(0x1BD11BDA))
    x0 = (u(x0) + ks[0]).astype(u)
    x1 = (u(x1) + ks[1]).astype(u)
    for i in range(5):
        for r in rots[i % 2]:
            x0 = (x0 + x1).astype(u)
            x1 = ((x1 << u(r)) | (x1 >> u(32 - r))).astype(u)
            x1 = x1 ^ x0
        x0 = (x0 + ks[(i + 1) % 3]).astype(u)
        x1 = (x1 + ks[(i + 2) % 3] + u(i + 1)).astype(u)
    return x0, x1


def _fold_in_key(k1, k2, data):
    return _np_threefry(k1, k2, 0, data)


with np.errstate(over="ignore"):
    _KEYS = tuple(_fold_in_key(0, 42, 3 * i) for i in range(N_STEPS))


def _threefry_gumbel(k1, k2, cnt):
    u32 = jnp.uint32
    ks = (u32(k1), u32(k2), u32(k1) ^ u32(k2) ^ u32(0x1BD11BDA))
    rots = ((13, 15, 26, 6), (17, 29, 16, 24))
    x1 = cnt + ks[1]
    x0 = x1 + ks[0]
    first = True
    for i in range(5):
        for r in rots[i % 2]:
            if first:
                first = False
            else:
                x0 = x0 + x1
            x1 = (x1 << u32(r)) | (x1 >> u32(32 - r))
            x1 = x1 ^ x0
        x0 = x0 + ks[(i + 1) % 3]
        x1 = x1 + ks[(i + 2) % 3] + u32(i + 1)
    bits = x0 ^ x1
    fb = (bits >> u32(9)) | u32(0x3F800000)
    f = jax.lax.bitcast_convert_type(fb, jnp.float32) - 1.0
    v = jnp.maximum(_TINY, f * (np.float32(1.0) - _TINY) + _TINY)
    return -jnp.log(-jnp.log(v))


def _body(x_ref, b_ref, w_ref, u0_ref, u1_ref, o_ref):
    f32 = jnp.float32
    u32 = jnp.uint32
    us = (u0_ref, u1_ref)
    W = w_ref[...]
    col_iota = jax.lax.broadcasted_iota(jnp.int32, (BB, CW), 1)
    ucnt = (jax.lax.broadcasted_iota(u32, (BB, CW), 0) * u32(DIM)
            + jax.lax.broadcasted_iota(u32, (BB, CW), 1))
    base_count = (pl.program_id(0) * (BB * DIM)).astype(u32)

    xw = jnp.dot(x_ref[...], W, preferred_element_type=f32)

    for i in range(N_STEPS):
        cur = x_ref if i == 0 else o_ref
        k1, k2 = _KEYS[i]
        u = us[i][:, 0]

        def pass_b(c, carry):
            mx, S, bv, bidx = carry
            c0 = c * CW
            xc = cur[:, pl.ds(c0, CW)]
            sgn = 1.0 - 2.0 * xc
            wc = w_ref[pl.ds(c0, CW), :]
            gx = jax.lax.dot_general(xw, wc, (((1,), (1,)), ((), ())),
                                     preferred_element_type=f32)
            fd = sgn * (gx + b_ref[:, pl.ds(c0, CW)]) * 0.5
            cmx = jnp.max(fd, axis=-1)
            nmx = jnp.maximum(mx, cmx)
            S = S * jnp.exp(mx - nmx) + jnp.sum(
                jnp.exp(fd - nmx[:, None]), axis=-1)
            g = _threefry_gumbel(k1, k2, ucnt + (base_count + u32(c0)))
            m = fd + g
            cbv = jnp.max(m, axis=-1)
            cbi = jnp.argmax(m, axis=-1) + c0
            upd = cbv > bv
            bv = jnp.where(upd, cbv, bv)
            bidx = jnp.where(upd, cbi, bidx)
            return nmx, S, bv, bidx

        carry = (jnp.full((BB,), _NEG_INF, f32), jnp.zeros((BB,), f32),
                 jnp.full((BB,), _NEG_INF, f32), jnp.zeros((BB,), jnp.int32))
        for c in range(NC):
            carry = pass_b(c, carry)
        mx, S, _, idx = carry
        lse_f = mx + jnp.log(S)

        def pass_g(c, carry):
            bi, s, Wi = carry
            c0 = c * CW
            mask = (col_iota + c0) == idx[:, None]
            xc = cur[:, pl.ds(c0, CW)]
            bi = bi + jnp.sum(jnp.where(mask, b_ref[:, pl.ds(c0, CW)], 0.0),
                              axis=-1)
            s = s + jnp.sum(jnp.where(mask, 1.0 - 2.0 * xc, 0.0), axis=-1)
            Wi = Wi + jnp.dot(mask.astype(f32), w_ref[pl.ds(c0, CW), :],
                              preferred_element_type=f32)
            return bi, s, Wi

        carry = (jnp.zeros((BB,), f32), jnp.zeros((BB,), f32),
                 jnp.zeros((BB, RANK), f32))
        for c in range(NC):
            carry = pass_g(c, carry)
        bi, s, Wi = carry

        xw_new = xw + s[:, None] * Wi
        fd_at = s * (jnp.sum(xw * Wi, axis=-1) + bi) * 0.5
        rd_at = -s * (jnp.sum(xw_new * Wi, axis=-1) + bi) * 0.5
        m_term = s * bi + 0.5 * (jnp.sum(xw_new * xw_new, axis=-1)
                                 - jnp.sum(xw * xw, axis=-1))

        def pass_c(c, carry):
            mxr, Sr = carry
            c0 = c * CW
            xc = cur[:, pl.ds(c0, CW)]
            t = 1.0 - 2.0 * xc
            mask = (col_iota + c0) == idx[:, None]
            sgn_new = jnp.where(mask, -t, t)
            wc = w_ref[pl.ds(c0, CW), :]
            gxn = jax.lax.dot_general(xw_new, wc, (((1,), (1,)), ((), ())),
                                      preferred_element_type=f32)
            rd = sgn_new * (gxn + b_ref[:, pl.ds(c0, CW)]) * 0.5
            cmx = jnp.max(rd, axis=-1)
            nmx = jnp.maximum(mxr, cmx)
            Sr = Sr * jnp.exp(mxr - nmx) + jnp.sum(
                jnp.exp(rd - nmx[:, None]), axis=-1)
            return nmx, Sr

        carry = (jnp.full((BB,), _NEG_INF, f32), jnp.zeros((BB,), f32))
        for c in range(NC):
            carry = pass_c(c, carry)
        mxr, Sr = carry
        lse_r = mxr + jnp.log(Sr)

        la = m_term + (rd_at - lse_r) - (fd_at - lse_f)
        a = jnp.exp(la) > u
        xw = jnp.where(a[:, None], xw_new, xw)

        def pass_u(c, _):
            c0 = c * CW
            mask = (col_iota + c0) == idx[:, None]
            xc = cur[:, pl.ds(c0, CW)]
            o_ref[:, pl.ds(c0, CW)] = jnp.where(
                mask & a[:, None], 1.0 - xc, xc)
            return 0

        for c in range(NC):
            pass_u(c, 0)


@jax.jit
def kernel(x, b, W):
    base = jax.random.key(42)
    us = []
    for i in range(N_STEPS):
        ka = jax.random.fold_in(base, 3 * i + 1)
        u = jax.random.uniform(ka, (BATCH,), jnp.float32)
        us.append(jnp.broadcast_to(u[:, None], (BATCH, 128)))
    b2 = b.reshape(1, DIM)

    grid = (BATCH // BB,)
    row = lambda i: (i, 0)
    fixed = lambda i: (0, 0)
    return pl.pallas_call(
        _body,
        grid=grid,
        in_specs=[
            pl.BlockSpec((BB, DIM), row),
            pl.BlockSpec((1, DIM), fixed),
            pl.BlockSpec((DIM, RANK), fixed),
            pl.BlockSpec((BB, 128), row),
            pl.BlockSpec((BB, 128), row),
        ],
        out_specs=pl.BlockSpec((BB, DIM), row),
        out_shape=jax.ShapeDtypeStruct((BATCH, DIM), x.dtype),
    )(x, b2, W, us[0], us[1])


# speedup vs baseline: 1.4591x; 1.0055x over previous
import jax
import jax.numpy as jnp
import numpy as np
from jax.experimental import pallas as pl

DIM = 32768
BATCH = 128
N_STEPS = 2
TEMP = 2.0
RANK = 64

BB = 16
_TINY = np.float32(np.finfo(np.float32).tiny)


def _np_threefry(k1, k2, x0, x1):
    u = np.uint32
    rots = ((13, 15, 26, 6), (17, 29, 16, 24))
    ks = (u(k1), u(k2), u(k1) ^ u(k2) ^ u(0x1BD11BDA))
    x0 = (u(x0) + ks[0]).astype(u)
    x1 = (u(x1) + ks[1]).astype(u)
    for i in range(5):
        for r in rots[i % 2]:
            x0 = (x0 + x1).astype(u)
            x1 = ((x1 << u(r)) | (x1 >> u(32 - r))).astype(u)
            x1 = x1 ^ x0
        x0 = (x0 + ks[(i + 1) % 3]).astype(u)
        x1 = (x1 + ks[(i + 2) % 3] + u(i + 1)).astype(u)
    return x0, x1


def _fold_in_key(k1, k2, data):
    return _np_threefry(k1, k2, 0, data)


with np.errstate(over="ignore"):
    _KEYS = tuple(_fold_in_key(0, 42, 3 * i) for i in range(N_STEPS))


def _threefry_gumbel(k1, k2, cnt):
    u32 = jnp.uint32
    ks = (u32(k1), u32(k2), u32(k1) ^ u32(k2) ^ u32(0x1BD11BDA))
    rots = ((13, 15, 26, 6), (17, 29, 16, 24))
    x1 = cnt + ks[1]
    x0 = x1 + ks[0]
    first = True
    for i in range(5):
        for r in rots[i % 2]:
            if first:
                first = False
            else:
                x0 = x0 + x1
            x1 = (x1 << u32(r)) | (x1 >> u32(32 - r))
            x1 = x1 ^ x0
        x0 = x0 + ks[(i + 1) % 3]
        x1 = x1 + ks[(i + 2) % 3] + u32(i + 1)
    bits = x0 ^ x1
    fb = (bits >> u32(9)) | u32(0x3F800000)
    f = jax.lax.bitcast_convert_type(fb, jnp.float32) - 1.0
    v = jnp.maximum(_TINY, f * (np.float32(1.0) - _TINY) + _TINY)
    return -jnp.log(-jnp.log(v))


def _body(x_ref, b_ref, w_ref, u0_ref, u1_ref, o_ref):
        x = x_ref[...]
        b = b_ref[...]
        W = w_ref[...]
        us = (u0_ref, u1_ref)

        f32 = jnp.float32
        xw = jnp.dot(x, W, preferred_element_type=f32)
        iota = jax.lax.broadcasted_iota(jnp.int32, (BB, DIM), 1)
        base_count = (pl.program_id(0) * (BB * DIM)).astype(jnp.uint32)
        u32 = jnp.uint32
        cnt = (jax.lax.broadcasted_iota(u32, (BB, DIM), 0) * u32(DIM)
               + jax.lax.broadcasted_iota(u32, (BB, DIM), 1) + base_count)
        sgn = 1.0 - 2.0 * x

        for i in range(N_STEPS):
            g = _threefry_gumbel(_KEYS[i][0], _KEYS[i][1], cnt)
            u = us[i][:, 0]

            gx = jax.lax.dot_general(xw, W, (((1,), (1,)), ((), ())),
                                     preferred_element_type=f32) + b
            fd = sgn * gx * 0.5
            mx = jnp.max(fd, axis=-1)
            lse_f = mx + jnp.log(jnp.sum(jnp.exp(fd - mx[:, None]), axis=-1))

            idx = jnp.argmax(fd + g, axis=-1)
            mask = iota == idx[:, None]

            bi = jnp.sum(jnp.where(mask, b, 0.0), axis=-1)
            s = jnp.sum(jnp.where(mask, sgn, 0.0), axis=-1)
            Wi = jnp.dot(mask.astype(f32), W, preferred_element_type=f32)
            xw_new = xw + s[:, None] * Wi
            sgn_new = jnp.where(mask, -sgn, sgn)
            fd_at = s * (jnp.sum(xw * Wi, axis=-1) + bi) * 0.5
            rd_at = -s * (jnp.sum(xw_new * Wi, axis=-1) + bi) * 0.5

            gx_new = jax.lax.dot_general(xw_new, W, (((1,), (1,)), ((), ())),
                                         preferred_element_type=f32) + b
            rd = sgn_new * gx_new * 0.5
            mxr = jnp.max(rd, axis=-1)
            lse_r = mxr + jnp.log(jnp.sum(jnp.exp(rd - mxr[:, None]), axis=-1))

            m_term = s * bi + 0.5 * (jnp.sum(xw_new * xw_new, axis=-1)
                                     - jnp.sum(xw * xw, axis=-1))
            la = m_term + (rd_at - lse_r) - (fd_at - lse_f)
            a = jnp.exp(la) > u
            flip = mask & a[:, None]
            x = jnp.where(flip, 1.0 - x, x)
            xw = jnp.where(a[:, None], xw_new, xw)
            if i + 1 < N_STEPS:
                sgn = jnp.where(flip, -sgn, sgn)

        o_ref[...] = x


@jax.jit
def kernel(x, b, W):
    base = jax.random.key(42)
    us = []
    for i in range(N_STEPS):
        ka = jax.random.fold_in(base, 3 * i + 1)
        u = jax.random.uniform(ka, (BATCH,), jnp.float32)
        us.append(jnp.broadcast_to(u[:, None], (BATCH, 128)))
    b2 = b.reshape(1, DIM)

    grid = (BATCH // BB,)
    row = lambda i: (i, 0)
    fixed = lambda i: (0, 0)
    return pl.pallas_call(
        _body,
        grid=grid,
        in_specs=[
            pl.BlockSpec((BB, DIM), row),
            pl.BlockSpec((1, DIM), fixed),
            pl.BlockSpec((DIM, RANK), fixed),
            pl.BlockSpec((BB, 128), row),
            pl.BlockSpec((BB, 128), row),
        ],
        out_specs=pl.BlockSpec((BB, DIM), row),
        out_shape=jax.ShapeDtypeStruct((BATCH, DIM), x.dtype),
    )(x, b2, W, us[0], us[1])


# speedup vs baseline: 1.5307x; 1.0490x over previous
import jax
import jax.numpy as jnp
import numpy as np
from jax.experimental import pallas as pl

DIM = 32768
BATCH = 128
N_STEPS = 2
TEMP = 2.0
RANK = 64

BB = 16
_TINY = np.float32(np.finfo(np.float32).tiny)


def _np_threefry(k1, k2, x0, x1):
    u = np.uint32
    rots = ((13, 15, 26, 6), (17, 29, 16, 24))
    ks = (u(k1), u(k2), u(k1) ^ u(k2) ^ u(0x1BD11BDA))
    x0 = (u(x0) + ks[0]).astype(u)
    x1 = (u(x1) + ks[1]).astype(u)
    for i in range(5):
        for r in rots[i % 2]:
            x0 = (x0 + x1).astype(u)
            x1 = ((x1 << u(r)) | (x1 >> u(32 - r))).astype(u)
            x1 = x1 ^ x0
        x0 = (x0 + ks[(i + 1) % 3]).astype(u)
        x1 = (x1 + ks[(i + 2) % 3] + u(i + 1)).astype(u)
    return x0, x1


def _fold_in_key(k1, k2, data):
    return _np_threefry(k1, k2, 0, data)


with np.errstate(over="ignore"):
    _KEYS = tuple(_fold_in_key(0, 42, 3 * i) for i in range(N_STEPS))


def _threefry_gumbel(k1, k2, cnt):
    u32 = jnp.uint32
    ks = (u32(k1), u32(k2), u32(k1) ^ u32(k2) ^ u32(0x1BD11BDA))
    rots = ((13, 15, 26, 6), (17, 29, 16, 24))
    x1 = cnt + ks[1]
    x0 = x1 + ks[0]
    first = True
    for i in range(5):
        for r in rots[i % 2]:
            if first:
                first = False
            else:
                x0 = x0 + x1
            x1 = (x1 << u32(r)) | (x1 >> u32(32 - r))
            x1 = x1 ^ x0
        x0 = x0 + ks[(i + 1) % 3]
        x1 = x1 + ks[(i + 2) % 3] + u32(i + 1)
    bits = x0 ^ x1
    fb = (bits >> u32(9)) | u32(0x3F800000)
    f = jax.lax.bitcast_convert_type(fb, jnp.float32) - 1.0
    v = jnp.maximum(_TINY, f * (np.float32(1.0) - _TINY) + _TINY)
    return -jnp.log(-jnp.log(v))


def _body(x_ref, b_ref, w_ref, u0_ref, u1_ref, o_ref):
        x = x_ref[...]
        b = b_ref[...]
        W = w_ref[...]
        us = (u0_ref, u1_ref)

        f32 = jnp.float32
        xw = jnp.dot(x, W, preferred_element_type=f32)
        iota = jax.lax.broadcasted_iota(jnp.int32, (BB, DIM), 1)
        base_count = (pl.program_id(0) * (BB * DIM)).astype(jnp.uint32)
        u32 = jnp.uint32
        cnt = (jax.lax.broadcasted_iota(u32, (BB, DIM), 0) * u32(DIM)
               + jax.lax.broadcasted_iota(u32, (BB, DIM), 1) + base_count)
        sgn = 1.0 - 2.0 * x

        for i in range(N_STEPS):
            g = _threefry_gumbel(_KEYS[i][0], _KEYS[i][1], cnt)
            u = us[i][:, 0]

            gx = jax.lax.dot_general(xw, W, (((1,), (1,)), ((), ())),
                                     preferred_element_type=f32) + b
            fd = sgn * gx * 0.5
            mx = jnp.max(fd, axis=-1)
            lse_f = mx + jnp.log(jnp.sum(jnp.exp(fd - mx[:, None]), axis=-1))

            idx = jnp.argmax(fd + g, axis=-1)
            mask = iota == idx[:, None]

            bi = jnp.sum(jnp.where(mask, b, 0.0), axis=-1)
            s = jnp.sum(jnp.where(mask, sgn, 0.0), axis=-1)
            Wi = jnp.dot(mask.astype(f32), W, preferred_element_type=f32)
            xw_new = xw + s[:, None] * Wi
            x_new = jnp.where(mask, 1.0 - x, x)
            sgn_new = jnp.where(mask, -sgn, sgn)
            fd_at = s * (jnp.sum(xw * Wi, axis=-1) + bi) * 0.5
            rd_at = -s * (jnp.sum(xw_new * Wi, axis=-1) + bi) * 0.5

            gx_new = jax.lax.dot_general(xw_new, W, (((1,), (1,)), ((), ())),
                                         preferred_element_type=f32) + b
            rd = sgn_new * gx_new * 0.5
            mxr = jnp.max(rd, axis=-1)
            lse_r = mxr + jnp.log(jnp.sum(jnp.exp(rd - mxr[:, None]), axis=-1))

            m_term = s * bi + 0.5 * (jnp.sum(xw_new * xw_new, axis=-1)
                                     - jnp.sum(xw * xw, axis=-1))
            la = m_term + (rd_at - lse_r) - (fd_at - lse_f)
            a = jnp.exp(la) > u
            x = jnp.where(a[:, None], x_new, x)
            sgn = jnp.where(a[:, None], sgn_new, sgn)
            xw = jnp.where(a[:, None], xw_new, xw)

        o_ref[...] = x


@jax.jit
def kernel(x, b, W):
    base = jax.random.key(42)
    us = []
    for i in range(N_STEPS):
        ka = jax.random.fold_in(base, 3 * i + 1)
        u = jax.random.uniform(ka, (BATCH,), jnp.float32)
        us.append(jnp.broadcast_to(u[:, None], (BATCH, 128)))
    b2 = b.reshape(1, DIM)

    grid = (BATCH // BB,)
    row = lambda i: (i, 0)
    fixed = lambda i: (0, 0)
    return pl.pallas_call(
        _body,
        grid=grid,
        in_specs=[
            pl.BlockSpec((BB, DIM), row),
            pl.BlockSpec((1, DIM), fixed),
            pl.BlockSpec((DIM, RANK), fixed),
            pl.BlockSpec((BB, 128), row),
            pl.BlockSpec((BB, 128), row),
        ],
        out_specs=pl.BlockSpec((BB, DIM), row),
        out_shape=jax.ShapeDtypeStruct((BATCH, DIM), x.dtype),
    )(x, b2, W, us[0], us[1])
